# Initial kernel scaffold; baseline (speedup 1.0000x reference)
import jax
import jax.numpy as jnp
from jax.experimental import pallas as pl


def kernel(batch_set_tensor, batch_inst_tensor, emb_table, W1, b1, W2, b2, W3, b3):
    raise NotImplementedError("write your pallas kernel here")


# speedup vs baseline: 2.1874x; 2.1874x over previous
import functools

import jax
import jax.numpy as jnp
from jax import lax
from jax.experimental import pallas as pl
from jax.experimental.pallas import tpu as pltpu
from jax.experimental.pallas import tpu_sc as plsc


def _make_sc_gather(num_rows, table_rows, dim):
    info = plsc.get_sparse_core_info()
    nc, ns = info.num_cores, info.num_subcores
    nw = nc * ns
    assert num_rows % (nw * 128) == 0
    rows_per_w = num_rows // nw
    n_streams = 6
    while (rows_per_w // 128) % n_streams:
        n_streams -= 1
    chunk = n_streams * 128
    n_chunks = rows_per_w // chunk

    mesh = plsc.VectorSubcoreMesh(core_axis_name="c", subcore_axis_name="s")

    @functools.partial(
        pl.kernel,
        mesh=mesh,
        out_type=jax.ShapeDtypeStruct((num_rows, dim), jnp.float32),
        scratch_types=[
            pltpu.VMEM((chunk,), jnp.int32),
            pltpu.VMEM((chunk, dim), jnp.float32),
            pltpu.SemaphoreType.DMA,
        ],
        compiler_params=pltpu.CompilerParams(use_tc_tiling_on_sc=False),
    )
    def gather_k(ids_hbm, table_hbm, out_hbm, idx_v, rows_v, sem):
        wid = lax.axis_index("s") * nc + lax.axis_index("c")
        base = wid * rows_per_w

        def body(g, carry):
            off = base + g * chunk
            pltpu.sync_copy(ids_hbm.at[pl.ds(off, chunk)], idx_v)
            copies = [
                pltpu.async_copy(
                    table_hbm.at[idx_v.at[pl.ds(j * 128, 128)]],
                    rows_v.at[pl.ds(j * 128, 128)],
                    sem)
                for j in range(n_streams)
            ]
            for cp in copies:
                cp.wait()
            pltpu.sync_copy(rows_v, out_hbm.at[pl.ds(off, chunk)])
            return carry

        lax.fori_loop(0, n_chunks, body, 0)

    return gather_k


def _tc_body(x_ref, ids_ref, w1_ref, b1_ref, w2_ref, b2_ref, w3_ref, b3_ref,
             o_set, o_uni, o_prd, acc_ref, h0_ref, *, n_tok):
    j = pl.program_id(1)
    x = x_ref[0]
    m = (ids_ref[0] != 0).astype(jnp.float32)
    h = jnp.dot(x, w1_ref[...], preferred_element_type=jnp.float32)
    h = jnp.maximum(h + b1_ref[...], 0.0) * m

    @pl.when(j == 0)
    def _():
        acc_ref[...] = h
        h0_ref[...] = h

    @pl.when(j > 0)
    def _():
        acc_ref[...] = acc_ref[...] + h

    @pl.when(j == n_tok - 1)
    def _():
        tot = acc_ref[...]
        setp = tot - h0_ref[...]
        w2 = w2_ref[...]
        b2 = b2_ref[...]
        w3 = w3_ref[...]
        b3 = b3_ref[0, 0]
        s_s = jnp.maximum(
            jnp.dot(setp, w2, preferred_element_type=jnp.float32) + b2, 0.0)
        s_u = jnp.maximum(
            jnp.dot(tot, w2, preferred_element_type=jnp.float32) + b2, 0.0)
        sc_s = jnp.sum(s_s * w3, axis=1, keepdims=True) + b3
        sc_u = jnp.sum(s_u * w3, axis=1, keepdims=True) + b3
        o_set[...] = sc_s
        o_uni[...] = sc_u
        o_prd[...] = 1.0 / (1.0 + jnp.exp(sc_s - sc_u))


def kernel(batch_set_tensor, batch_inst_tensor, emb_table, W1, b1, W2, b2, W3, b3):
    B, L = batch_set_tensor.shape
    n_tok = L + 1
    D = emb_table.shape[1]
    H = W1.shape[1]
    T = B * n_tok

    union = jnp.concatenate([batch_inst_tensor, batch_set_tensor], axis=1)
    ids_t = union.T
    ids_flat = ids_t.reshape(T).astype(jnp.int32)

    gathered = _make_sc_gather(T, emb_table.shape[0], D)(ids_flat, emb_table)
    g3 = gathered.reshape(n_tok, B, D)
    ids3 = ids_t[:, :, None]

    BBLK = 1024
    nb = B // BBLK

    out_sds = jax.ShapeDtypeStruct((B, 1), jnp.float32)
    body = functools.partial(_tc_body, n_tok=n_tok)
    o_set, o_uni, o_prd = pl.pallas_call(
        body,
        grid=(nb, n_tok),
        in_specs=[
            pl.BlockSpec((1, BBLK, D), lambda i, j: (j, i, 0)),
            pl.BlockSpec((1, BBLK, 1), lambda i, j: (j, i, 0)),
            pl.BlockSpec((D, H), lambda i, j: (0, 0)),
            pl.BlockSpec((1, H), lambda i, j: (0, 0)),
            pl.BlockSpec((H, H), lambda i, j: (0, 0)),
            pl.BlockSpec((1, H), lambda i, j: (0, 0)),
            pl.BlockSpec((1, H), lambda i, j: (0, 0)),
            pl.BlockSpec((1, 1), lambda i, j: (0, 0)),
        ],
        out_specs=[
            pl.BlockSpec((BBLK, 1), lambda i, j: (i, 0)),
            pl.BlockSpec((BBLK, 1), lambda i, j: (i, 0)),
            pl.BlockSpec((BBLK, 1), lambda i, j: (i, 0)),
        ],
        out_shape=[out_sds, out_sds, out_sds],
        scratch_shapes=[
            pltpu.VMEM((BBLK, H), jnp.float32),
            pltpu.VMEM((BBLK, H), jnp.float32),
        ],
        compiler_params=pltpu.CompilerParams(
            dimension_semantics=("parallel", "arbitrary")),
    )(g3, ids3, W1, b1.reshape(1, H), W2, b2.reshape(1, H),
      W3.reshape(1, H), b3.reshape(1, 1))

    return (o_set, o_uni, o_prd)


# speedup vs baseline: 2.4424x; 1.1166x over previous
import functools

import jax
import jax.numpy as jnp
from jax import lax
from jax.experimental import pallas as pl
from jax.experimental.pallas import tpu as pltpu
from jax.experimental.pallas import tpu_sc as plsc


def _make_sc_gather(num_rows, dim):
    info = plsc.get_sparse_core_info()
    nc, ns = info.num_cores, info.num_subcores
    nw = nc * ns
    assert num_rows % (nw * 128) == 0
    rows_per_w = num_rows // nw
    n_streams = 6
    while (rows_per_w // 128) % n_streams:
        n_streams -= 1
    chunk = n_streams * 128
    n_chunks = rows_per_w // chunk

    mesh = plsc.VectorSubcoreMesh(core_axis_name="c", subcore_axis_name="s")

    @functools.partial(
        pl.kernel,
        mesh=mesh,
        out_type=jax.ShapeDtypeStruct((num_rows // 2, 2 * dim), jnp.float32),
        scratch_types=[
            pltpu.VMEM((chunk,), jnp.int32),
            pltpu.VMEM((chunk, dim), jnp.float32),
            pltpu.SemaphoreType.DMA,
        ],
        compiler_params=pltpu.CompilerParams(use_tc_tiling_on_sc=False),
    )
    def gather_k(ids_hbm, table_hbm, out_hbm, idx_v, rows_v, sem):
        wid = lax.axis_index("s") * nc + lax.axis_index("c")
        base = wid * rows_per_w

        def body(g, carry):
            off = base + g * chunk
            pltpu.sync_copy(ids_hbm.at[pl.ds(off, chunk)], idx_v)
            copies = [
                pltpu.async_copy(
                    table_hbm.at[idx_v.at[pl.ds(s * 128, 128)]],
                    rows_v.at[pl.ds(s * 128, 128)],
                    sem)
                for s in range(n_streams)
            ]
            for cp in copies:
                cp.wait()
            out_rows = out_hbm.at[pl.ds(off // 2, chunk // 2)]
            writes = [
                pltpu.async_copy(
                    rows_v.at[pl.ds((u * 2 + h) * 128, 128)],
                    out_rows.at[pl.ds(u * 128, 128), pl.ds(h * dim, dim)],
                    sem)
                for u in range(n_streams // 2) for h in range(2)
            ]
            for cp in writes:
                cp.wait()
            return carry

        lax.fori_loop(0, n_chunks, body, 0)

    return gather_k


def _tc_body(x_ref, ids_ref, w1_ref, b1_ref, w2_ref, b2_ref, w3_ref, b3_ref,
             o_set, o_uni, o_prd, acc_a, acc_b, h0_a, h0_b, *, n_tok, dim):
    j = pl.program_id(1)
    x2 = x_ref[...]
    ids = ids_ref[0]
    w1 = w1_ref[...]
    b1 = b1_ref[...]

    def half(xh, mh):
        h = jnp.dot(xh.astype(jnp.bfloat16), w1,
                    preferred_element_type=jnp.float32)
        return jnp.maximum(h + b1, 0.0) * mh

    h_a = half(x2[:, :dim], (ids[:, 0:1] != 0).astype(jnp.float32))
    h_b = half(x2[:, dim:], (ids[:, 1:2] != 0).astype(jnp.float32))

    @pl.when(j == 0)
    def _():
        acc_a[...] = h_a
        acc_b[...] = h_b
        h0_a[...] = h_a
        h0_b[...] = h_b

    @pl.when(j > 0)
    def _():
        acc_a[...] = acc_a[...] + h_a
        acc_b[...] = acc_b[...] + h_b

    @pl.when(j == n_tok - 1)
    def _():
        w2 = w2_ref[...]
        b2 = b2_ref[...]
        w3 = w3_ref[...]
        b3 = b3_ref[0, 0]

        def score(p):
            s = jnp.maximum(
                jnp.dot(p, w2, preferred_element_type=jnp.float32) + b2, 0.0)
            return jnp.sum(s * w3, axis=1, keepdims=True) + b3

        tot_a = acc_a[...]
        tot_b = acc_b[...]
        sc_s_a = score(tot_a - h0_a[...])
        sc_s_b = score(tot_b - h0_b[...])
        sc_u_a = score(tot_a)
        sc_u_b = score(tot_b)
        sc_s = jnp.concatenate([sc_s_a, sc_s_b], axis=1)
        sc_u = jnp.concatenate([sc_u_a, sc_u_b], axis=1)
        o_set[...] = sc_s
        o_uni[...] = sc_u
        o_prd[...] = 1.0 / (1.0 + jnp.exp(sc_s - sc_u))


def kernel(batch_set_tensor, batch_inst_tensor, emb_table, W1, b1, W2, b2, W3, b3):
    B, L = batch_set_tensor.shape
    n_tok = L + 1
    D = emb_table.shape[1]
    H = W1.shape[1]
    T = B * n_tok

    union = jnp.concatenate([batch_inst_tensor, batch_set_tensor], axis=1)
    ids_t = union.T
    ids_flat = ids_t.reshape(T).astype(jnp.int32)
    ids_shuf = ids_flat.reshape(T // 256, 128, 2).transpose(0, 2, 1).reshape(T)

    gathered = _make_sc_gather(T, D)(ids_shuf, emb_table)
    ids_p = ids_t.reshape(n_tok, B // 2, 2)

    BBLK = 1024
    HB = BBLK // 2
    nb = B // BBLK

    out_sds = jax.ShapeDtypeStruct((B // 2, 2), jnp.float32)
    body = functools.partial(_tc_body, n_tok=n_tok, dim=D)
    o_set, o_uni, o_prd = pl.pallas_call(
        body,
        grid=(nb, n_tok),
        in_specs=[
            pl.BlockSpec((HB, 2 * D), lambda i, j, nb=nb: (j * nb + i, 0)),
            pl.BlockSpec((1, HB, 2), lambda i, j: (j, i, 0)),
            pl.BlockSpec((D, H), lambda i, j: (0, 0)),
            pl.BlockSpec((1, H), lambda i, j: (0, 0)),
            pl.BlockSpec((H, H), lambda i, j: (0, 0)),
            pl.BlockSpec((1, H), lambda i, j: (0, 0)),
            pl.BlockSpec((1, H), lambda i, j: (0, 0)),
            pl.BlockSpec((1, 1), lambda i, j: (0, 0)),
        ],
        out_specs=[
            pl.BlockSpec((HB, 2), lambda i, j: (i, 0)),
            pl.BlockSpec((HB, 2), lambda i, j: (i, 0)),
            pl.BlockSpec((HB, 2), lambda i, j: (i, 0)),
        ],
        out_shape=[out_sds, out_sds, out_sds],
        scratch_shapes=[
            pltpu.VMEM((HB, H), jnp.float32),
            pltpu.VMEM((HB, H), jnp.float32),
            pltpu.VMEM((HB, H), jnp.float32),
            pltpu.VMEM((HB, H), jnp.float32),
        ],
        compiler_params=pltpu.CompilerParams(
            dimension_semantics=("parallel", "arbitrary")),
    )(gathered, ids_p, W1.astype(jnp.bfloat16), b1.reshape(1, H),
      W2, b2.reshape(1, H), W3.reshape(1, H), b3.reshape(1, 1))

    return (o_set.reshape(B, 1), o_uni.reshape(B, 1), o_prd.reshape(B, 1))


# speedup vs baseline: 2.9511x; 1.2083x over previous
import functools

import jax
import jax.numpy as jnp
from jax import lax
from jax.experimental import pallas as pl
from jax.experimental.pallas import tpu as pltpu
from jax.experimental.pallas import tpu_sc as plsc


def _make_sc_gather(num_rows, dim):
    info = plsc.get_sparse_core_info()
    nc, ns = info.num_cores, info.num_subcores
    nw = nc * ns
    assert num_rows % (nw * 256) == 0
    rows_per_w = num_rows // nw
    n_streams = 6
    while (rows_per_w // 128) % n_streams or n_streams % 2:
        n_streams -= 1
    chunk = n_streams * 128
    n_chunks = rows_per_w // chunk

    mesh = plsc.VectorSubcoreMesh(core_axis_name="c", subcore_axis_name="s")

    @functools.partial(
        pl.kernel,
        mesh=mesh,
        out_type=jax.ShapeDtypeStruct((num_rows // 2, 2 * dim), jnp.float32),
        scratch_types=[
            pltpu.VMEM((chunk,), jnp.int32),
            pltpu.VMEM((chunk, dim), jnp.float32),
            pltpu.SemaphoreType.DMA,
        ],
        compiler_params=pltpu.CompilerParams(use_tc_tiling_on_sc=False),
    )
    def gather_k(ids_hbm, table_hbm, out_hbm, idx_v, rows_v, sem):
        wid = lax.axis_index("s") * nc + lax.axis_index("c")
        base = wid * rows_per_w

        def body(g, carry):
            off = base + g * chunk
            pltpu.sync_copy(ids_hbm.at[pl.ds(off, chunk)], idx_v)
            copies = [
                pltpu.async_copy(
                    table_hbm.at[idx_v.at[pl.ds(s * 128, 128)]],
                    rows_v.at[pl.ds(s * 128, 128)],
                    sem)
                for s in range(n_streams)
            ]
            for cp in copies:
                cp.wait()
            out_rows = out_hbm.at[pl.ds(off // 2, chunk // 2)]
            writes = [
                pltpu.async_copy(
                    rows_v.at[pl.ds((u * 2 + h) * 128, 128)],
                    out_rows.at[pl.ds(u * 128, 128), pl.ds(h * dim, dim)],
                    sem)
                for u in range(n_streams // 2) for h in range(2)
            ]
            for cp in writes:
                cp.wait()
            return carry

        lax.fori_loop(0, n_chunks, body, 0)

    return gather_k


def _tc_body(x_ref, aux_ref, e0_ref, w1_ref, b1_ref, w2_ref, b2_ref,
             w3_ref, b3_ref, o_set, o_uni, o_prd, acc_a, acc_b,
             *, n_tok, dim):
    j = pl.program_id(1)
    x2 = x_ref[...]
    w1 = w1_ref[...]
    b1 = b1_ref[...]

    def half(xh):
        h = jnp.dot(xh.astype(jnp.bfloat16), w1,
                    preferred_element_type=jnp.float32)
        return jnp.maximum(h + b1, 0.0)

    h_a = half(x2[:, :dim])
    h_b = half(x2[:, dim:])

    @pl.when(j == 0)
    def _():
        acc_a[...] = h_a
        acc_b[...] = h_b

    @pl.when(jnp.logical_and(j > 0, j < n_tok - 1))
    def _():
        acc_a[...] = acc_a[...] + h_a
        acc_b[...] = acc_b[...] + h_b

    @pl.when(j == n_tok - 1)
    def _():
        c = jnp.maximum(
            jnp.dot(e0_ref[...].astype(jnp.bfloat16), w1,
                    preferred_element_type=jnp.float32) + b1, 0.0)
        aux = aux_ref[...]
        w2 = w2_ref[...]
        b2 = b2_ref[...]
        w3 = w3_ref[...]
        b3 = b3_ref[0, 0]

        def score(p):
            s = jnp.maximum(
                jnp.dot(p.astype(jnp.bfloat16), w2,
                        preferred_element_type=jnp.float32) + b2, 0.0)
            return jnp.sum(s * w3, axis=1, keepdims=True) + b3

        sc_s, sc_u = [], []
        for acc, h_i, k in ((acc_a, h_a, 0), (acc_b, h_b, 1)):
            set_p = acc[...] - aux[:, k:k + 1] * c
            uni_p = set_p + h_i - aux[:, 2 + k:3 + k] * c
            sc_s.append(score(set_p))
            sc_u.append(score(uni_p))
        sc_s = jnp.concatenate(sc_s, axis=1)
        sc_u = jnp.concatenate(sc_u, axis=1)
        o_set[...] = sc_s
        o_uni[...] = sc_u
        o_prd[...] = 1.0 / (1.0 + jnp.exp(sc_s - sc_u))


def kernel(batch_set_tensor, batch_inst_tensor, emb_table, W1, b1, W2, b2, W3, b3):
    B, L = batch_set_tensor.shape
    n_tok = L + 1
    D = emb_table.shape[1]
    H = W1.shape[1]
    T = B * n_tok

    union = jnp.concatenate([batch_set_tensor, batch_inst_tensor], axis=1)
    ids_t = union.T
    ids_flat = ids_t.reshape(T).astype(jnp.int32)
    ids_shuf = ids_flat.reshape(T // 256, 128, 2).transpose(0, 2, 1).reshape(T)

    gathered = _make_sc_gather(T, D)(ids_shuf, emb_table)

    n0s = jnp.sum(batch_set_tensor == 0, axis=1).astype(jnp.float32)
    i0 = (batch_inst_tensor[:, 0] == 0).astype(jnp.float32)
    aux = jnp.concatenate(
        [n0s.reshape(B // 2, 2), i0.reshape(B // 2, 2)], axis=1)
    e0 = emb_table[0:1, :]

    BBLK = 2048
    HB = BBLK // 2
    nb = B // BBLK

    out_sds = jax.ShapeDtypeStruct((B // 2, 2), jnp.float32)
    body = functools.partial(_tc_body, n_tok=n_tok, dim=D)
    o_set, o_uni, o_prd = pl.pallas_call(
        body,
        grid=(nb, n_tok),
        in_specs=[
            pl.BlockSpec((HB, 2 * D), lambda i, j, nb=nb: (j * nb + i, 0)),
            pl.BlockSpec((HB, 4), lambda i, j: (i, 0)),
            pl.BlockSpec((1, D), lambda i, j: (0, 0)),
            pl.BlockSpec((D, H), lambda i, j: (0, 0)),
            pl.BlockSpec((1, H), lambda i, j: (0, 0)),
            pl.BlockSpec((H, H), lambda i, j: (0, 0)),
            pl.BlockSpec((1, H), lambda i, j: (0, 0)),
            pl.BlockSpec((1, H), lambda i, j: (0, 0)),
            pl.BlockSpec((1, 1), lambda i, j: (0, 0)),
        ],
        out_specs=[
            pl.BlockSpec((HB, 2), lambda i, j: (i, 0)),
            pl.BlockSpec((HB, 2), lambda i, j: (i, 0)),
            pl.BlockSpec((HB, 2), lambda i, j: (i, 0)),
        ],
        out_shape=[out_sds, out_sds, out_sds],
        scratch_shapes=[
            pltpu.VMEM((HB, H), jnp.float32),
            pltpu.VMEM((HB, H), jnp.float32),
        ],
        compiler_params=pltpu.CompilerParams(
            dimension_semantics=("parallel", "arbitrary")),
    )(gathered, aux, e0, W1.astype(jnp.bfloat16), b1.reshape(1, H),
      W2.astype(jnp.bfloat16), b2.reshape(1, H), W3.reshape(1, H),
      b3.reshape(1, 1))

    return (o_set.reshape(B, 1), o_uni.reshape(B, 1), o_prd.reshape(B, 1))


# speedup vs baseline: 2.9621x; 1.0037x over previous
import functools

import jax
import jax.numpy as jnp
from jax import lax
from jax.experimental import pallas as pl
from jax.experimental.pallas import tpu as pltpu
from jax.experimental.pallas import tpu_sc as plsc


def _make_sc_gather(num_rows, dim):
    info = plsc.get_sparse_core_info()
    nc, ns = info.num_cores, info.num_subcores
    nw = nc * ns
    assert num_rows % (nw * 256) == 0
    rows_per_w = num_rows // nw
    n_streams = 6
    while (rows_per_w // 128) % n_streams or n_streams % 2:
        n_streams -= 1
    chunk = n_streams * 128
    n_chunks = rows_per_w // chunk

    mesh = plsc.VectorSubcoreMesh(core_axis_name="c", subcore_axis_name="s")

    @functools.partial(
        pl.kernel,
        mesh=mesh,
        out_type=jax.ShapeDtypeStruct((num_rows // 2, 2 * dim), jnp.float32),
        scratch_types=[
            pltpu.VMEM((chunk,), jnp.int32),
            pltpu.VMEM((chunk, dim), jnp.float32),
            pltpu.SemaphoreType.DMA,
        ],
        compiler_params=pltpu.CompilerParams(use_tc_tiling_on_sc=False),
    )
    def gather_k(ids_hbm, table_hbm, out_hbm, idx_v, rows_v, sem):
        wid = lax.axis_index("s") * nc + lax.axis_index("c")
        base = wid * rows_per_w

        def body(g, carry):
            off = base + g * chunk
            pltpu.sync_copy(ids_hbm.at[pl.ds(off, chunk)], idx_v)
            copies = [
                pltpu.async_copy(
                    table_hbm.at[idx_v.at[pl.ds(s * 128, 128)]],
                    rows_v.at[pl.ds(s * 128, 128)],
                    sem)
                for s in range(n_streams)
            ]
            for cp in copies:
                cp.wait()
            out_rows = out_hbm.at[pl.ds(off // 2, chunk // 2)]
            writes = [
                pltpu.async_copy(
                    rows_v.at[pl.ds((u * 2 + h) * 128, 128)],
                    out_rows.at[pl.ds(u * 128, 128), pl.ds(h * dim, dim)],
                    sem)
                for u in range(n_streams // 2) for h in range(2)
            ]
            for cp in writes:
                cp.wait()
            return carry

        lax.fori_loop(0, n_chunks, body, 0)

    return gather_k


def _tc_body(x_ref, aux_ref, e0_ref, w1d_ref, b1d_ref, w2d_ref, b2d_ref,
             sel3_ref, b3_ref, o_set, o_uni, o_prd, acc,
             *, n_tok, dim, hid):
    j = pl.program_id(1)
    x2 = x_ref[...]
    w1d = w1d_ref[...]
    h = jnp.maximum(
        jnp.dot(x2.astype(jnp.bfloat16), w1d,
                preferred_element_type=jnp.float32) + b1d_ref[...], 0.0)

    @pl.when(j == 0)
    def _():
        acc[...] = h

    @pl.when(jnp.logical_and(j > 0, j < n_tok - 1))
    def _():
        acc[...] = acc[...] + h

    @pl.when(j == n_tok - 1)
    def _():
        cd = jnp.maximum(
            jnp.dot(e0_ref[...].astype(jnp.bfloat16), w1d,
                    preferred_element_type=jnp.float32) + b1d_ref[...],
            0.0)
        aux = aux_ref[...]
        lane = lax.broadcasted_iota(jnp.int32, (1, 2 * hid), 1)
        in_a = lane < hid
        n0 = jnp.where(in_a, aux[:, 0:1], aux[:, 1:2])
        i0 = jnp.where(in_a, aux[:, 2:3], aux[:, 3:4])
        set_p = acc[...] - n0 * cd
        uni_p = set_p + h - i0 * cd
        w2d = w2d_ref[...]
        b2d = b2d_ref[...]
        sel3 = sel3_ref[...]
        b3 = b3_ref[0, 0]

        def score(p):
            s = jnp.maximum(
                jnp.dot(p.astype(jnp.bfloat16), w2d,
                        preferred_element_type=jnp.float32) + b2d, 0.0)
            return jnp.dot(s.astype(jnp.bfloat16), sel3,
                           preferred_element_type=jnp.float32) + b3

        sc_s = score(set_p)
        sc_u = score(uni_p)
        o_set[...] = sc_s
        o_uni[...] = sc_u
        o_prd[...] = 1.0 / (1.0 + jnp.exp(sc_s - sc_u))


def kernel(batch_set_tensor, batch_inst_tensor, emb_table, W1, b1, W2, b2, W3, b3):
    B, L = batch_set_tensor.shape
    n_tok = L + 1
    D = emb_table.shape[1]
    H = W1.shape[1]
    T = B * n_tok

    union = jnp.concatenate([batch_set_tensor, batch_inst_tensor], axis=1)
    ids_t = union.T
    ids_flat = ids_t.reshape(T).astype(jnp.int32)
    ids_shuf = ids_flat.reshape(T // 256, 128, 2).transpose(0, 2, 1).reshape(T)

    gathered = _make_sc_gather(T, D)(ids_shuf, emb_table)

    n0s = jnp.sum(batch_set_tensor == 0, axis=1).astype(jnp.float32)
    i0 = (batch_inst_tensor[:, 0] == 0).astype(jnp.float32)
    aux = jnp.concatenate(
        [n0s.reshape(B // 2, 2), i0.reshape(B // 2, 2)], axis=1)
    e0 = emb_table[0:1, :]

    BBLK = 2048
    HB = BBLK // 2
    nb = B // BBLK

    zdh = jnp.zeros((D, H), jnp.float32)
    zhh = jnp.zeros((H, H), jnp.float32)
    zh1 = jnp.zeros((H, 1), jnp.float32)
    w1d = jnp.concatenate([
        jnp.concatenate([W1, zdh], axis=1),
        jnp.concatenate([zdh, W1], axis=1)], axis=0).astype(jnp.bfloat16)
    w2d = jnp.concatenate([
        jnp.concatenate([W2, zhh], axis=1),
        jnp.concatenate([zhh, W2], axis=1)], axis=0).astype(jnp.bfloat16)
    sel3 = jnp.concatenate([
        jnp.concatenate([W3, zh1], axis=1),
        jnp.concatenate([zh1, W3], axis=1)], axis=0).astype(jnp.bfloat16)
    b1d = jnp.concatenate([b1, b1]).reshape(1, 2 * H)
    b2d = jnp.concatenate([b2, b2]).reshape(1, 2 * H)

    out_sds = jax.ShapeDtypeStruct((B // 2, 2), jnp.float32)
    body = functools.partial(_tc_body, n_tok=n_tok, dim=D, hid=H)
    o_set, o_uni, o_prd = pl.pallas_call(
        body,
        grid=(nb, n_tok),
        in_specs=[
            pl.BlockSpec((HB, 2 * D), lambda i, j, nb=nb: (j * nb + i, 0)),
            pl.BlockSpec((HB, 4), lambda i, j: (i, 0)),
            pl.BlockSpec((1, 2 * D), lambda i, j: (0, 0)),
            pl.BlockSpec((2 * D, 2 * H), lambda i, j: (0, 0)),
            pl.BlockSpec((1, 2 * H), lambda i, j: (0, 0)),
            pl.BlockSpec((2 * H, 2 * H), lambda i, j: (0, 0)),
            pl.BlockSpec((1, 2 * H), lambda i, j: (0, 0)),
            pl.BlockSpec((2 * H, 2), lambda i, j: (0, 0)),
            pl.BlockSpec((1, 1), lambda i, j: (0, 0)),
        ],
        out_specs=[
            pl.BlockSpec((HB, 2), lambda i, j: (i, 0)),
            pl.BlockSpec((HB, 2), lambda i, j: (i, 0)),
            pl.BlockSpec((HB, 2), lambda i, j: (i, 0)),
        ],
        out_shape=[out_sds, out_sds, out_sds],
        scratch_shapes=[
            pltpu.VMEM((HB, 2 * H), jnp.float32),
        ],
        compiler_params=pltpu.CompilerParams(
            dimension_semantics=("parallel", "arbitrary")),
    )(gathered, aux, jnp.concatenate([e0, e0], axis=1), w1d, b1d,
      w2d, b2d, sel3, b3.reshape(1, 1))

    return (o_set.reshape(B, 1), o_uni.reshape(B, 1), o_prd.reshape(B, 1))


# speedup vs baseline: 3.1939x; 1.0783x over previous
import functools

import jax
import jax.numpy as jnp
from jax import lax
from jax.experimental import pallas as pl
from jax.experimental.pallas import tpu as pltpu
from jax.experimental.pallas import tpu_sc as plsc


def _make_sc_gather(num_rows, dim):
    info = plsc.get_sparse_core_info()
    nc, ns = info.num_cores, info.num_subcores
    nw = nc * ns
    assert num_rows % (nw * 256) == 0
    rows_per_w = num_rows // nw
    n_streams = 6
    while (rows_per_w // 128) % n_streams or n_streams % 2:
        n_streams -= 1
    chunk = n_streams * 128
    n_chunks = rows_per_w // chunk

    mesh = plsc.VectorSubcoreMesh(core_axis_name="c", subcore_axis_name="s")

    @functools.partial(
        pl.kernel,
        mesh=mesh,
        out_type=jax.ShapeDtypeStruct((num_rows // 2, 2 * dim), jnp.float32),
        scratch_types=[
            pltpu.VMEM((chunk,), jnp.int32),
            pltpu.VMEM((chunk, dim), jnp.float32),
            pltpu.SemaphoreType.DMA,
        ],
        compiler_params=pltpu.CompilerParams(use_tc_tiling_on_sc=False),
    )
    def gather_k(ids_hbm, table_hbm, out_hbm, idx_v, rows_v, sem):
        wid = lax.axis_index("s") * nc + lax.axis_index("c")
        base = wid * rows_per_w

        def body(g, carry):
            off = base + g * chunk
            pltpu.sync_copy(ids_hbm.at[pl.ds(off, chunk)], idx_v)
            copies = [
                pltpu.async_copy(
                    table_hbm.at[idx_v.at[pl.ds(s * 128, 128)]],
                    rows_v.at[pl.ds(s * 128, 128)],
                    sem)
                for s in range(n_streams)
            ]
            for cp in copies:
                cp.wait()
            out_rows = out_hbm.at[pl.ds(off // 2, chunk // 2)]
            writes = [
                pltpu.async_copy(
                    rows_v.at[pl.ds((u * 2 + h) * 128, 128)],
                    out_rows.at[pl.ds(u * 128, 128), pl.ds(h * dim, dim)],
                    sem)
                for u in range(n_streams // 2) for h in range(2)
            ]
            for cp in writes:
                cp.wait()
            return carry

        lax.fori_loop(0, n_chunks, body, 0)

    return gather_k


def _tc_body(x_ref, aux_ref, e0_ref, w1d_ref, b1d_ref, w2d_ref, b2d_ref,
             sel3_ref, b3_ref, o_set, o_uni, o_prd, acc,
             *, n_tok, dim, hid):
    j = pl.program_id(1)
    x2 = x_ref[...]
    w1d = w1d_ref[...]
    h = jnp.maximum(
        jnp.dot(x2.astype(jnp.bfloat16), w1d,
                preferred_element_type=jnp.float32) + b1d_ref[...], 0.0)

    @pl.when(j == 0)
    def _():
        acc[...] = h

    @pl.when(jnp.logical_and(j > 0, j < n_tok - 1))
    def _():
        acc[...] = acc[...] + h

    @pl.when(j == n_tok - 1)
    def _():
        cd = jnp.maximum(
            jnp.dot(e0_ref[...].astype(jnp.bfloat16), w1d,
                    preferred_element_type=jnp.float32) + b1d_ref[...],
            0.0)
        aux = aux_ref[...]
        lane = lax.broadcasted_iota(jnp.int32, (1, 2 * hid), 1)
        in_a = lane < hid
        n0 = jnp.where(in_a, aux[:, 0:1], aux[:, 1:2])
        i0 = jnp.where(in_a, aux[:, 2:3], aux[:, 3:4])
        set_p = acc[...] - n0 * cd
        uni_p = set_p + h - i0 * cd
        w2d = w2d_ref[...]
        b2d = b2d_ref[...]
        sel3 = sel3_ref[...]
        b3 = b3_ref[0, 0]

        def score(p):
            s = jnp.maximum(
                jnp.dot(p.astype(jnp.bfloat16), w2d,
                        preferred_element_type=jnp.float32) + b2d, 0.0)
            return jnp.dot(s.astype(jnp.bfloat16), sel3,
                           preferred_element_type=jnp.float32) + b3

        sc_s = score(set_p)
        sc_u = score(uni_p)
        o_set[...] = sc_s
        o_uni[...] = sc_u
        o_prd[...] = 1.0 / (1.0 + jnp.exp(sc_s - sc_u))


def kernel(batch_set_tensor, batch_inst_tensor, emb_table, W1, b1, W2, b2, W3, b3):
    B, L = batch_set_tensor.shape
    n_tok = L + 1
    D = emb_table.shape[1]
    H = W1.shape[1]
    T = B * n_tok

    union = jnp.concatenate([batch_set_tensor, batch_inst_tensor], axis=1)
    ids_t = union.T
    ids_flat = ids_t.reshape(T).astype(jnp.int32)
    ids_shuf = ids_flat.reshape(T // 256, 128, 2).transpose(0, 2, 1).reshape(T)

    gathered = _make_sc_gather(T, D)(ids_shuf, emb_table)

    n0s = jnp.sum(batch_set_tensor == 0, axis=1).astype(jnp.float32)
    i0 = (batch_inst_tensor[:, 0] == 0).astype(jnp.float32)
    aux = jnp.concatenate(
        [n0s.reshape(B // 2, 2), i0.reshape(B // 2, 2)], axis=1)
    e0 = emb_table[0:1, :]

    BBLK = 4096
    HB = BBLK // 2
    nb = B // BBLK

    zdh = jnp.zeros((D, H), jnp.float32)
    zhh = jnp.zeros((H, H), jnp.float32)
    zh1 = jnp.zeros((H, 1), jnp.float32)
    w1d = jnp.concatenate([
        jnp.concatenate([W1, zdh], axis=1),
        jnp.concatenate([zdh, W1], axis=1)], axis=0).astype(jnp.bfloat16)
    w2d = jnp.concatenate([
        jnp.concatenate([W2, zhh], axis=1),
        jnp.concatenate([zhh, W2], axis=1)], axis=0).astype(jnp.bfloat16)
    sel3 = jnp.concatenate([
        jnp.concatenate([W3, zh1], axis=1),
        jnp.concatenate([zh1, W3], axis=1)], axis=0).astype(jnp.bfloat16)
    b1d = jnp.concatenate([b1, b1]).reshape(1, 2 * H)
    b2d = jnp.concatenate([b2, b2]).reshape(1, 2 * H)

    out_sds = jax.ShapeDtypeStruct((B // 2, 2), jnp.float32)
    body = functools.partial(_tc_body, n_tok=n_tok, dim=D, hid=H)
    o_set, o_uni, o_prd = pl.pallas_call(
        body,
        grid=(nb, n_tok),
        in_specs=[
            pl.BlockSpec((HB, 2 * D), lambda i, j, nb=nb: (j * nb + i, 0)),
            pl.BlockSpec((HB, 4), lambda i, j: (i, 0)),
            pl.BlockSpec((1, 2 * D), lambda i, j: (0, 0)),
            pl.BlockSpec((2 * D, 2 * H), lambda i, j: (0, 0)),
            pl.BlockSpec((1, 2 * H), lambda i, j: (0, 0)),
            pl.BlockSpec((2 * H, 2 * H), lambda i, j: (0, 0)),
            pl.BlockSpec((1, 2 * H), lambda i, j: (0, 0)),
            pl.BlockSpec((2 * H, 2), lambda i, j: (0, 0)),
            pl.BlockSpec((1, 1), lambda i, j: (0, 0)),
        ],
        out_specs=[
            pl.BlockSpec((HB, 2), lambda i, j: (i, 0)),
            pl.BlockSpec((HB, 2), lambda i, j: (i, 0)),
            pl.BlockSpec((HB, 2), lambda i, j: (i, 0)),
        ],
        out_shape=[out_sds, out_sds, out_sds],
        scratch_shapes=[
            pltpu.VMEM((HB, 2 * H), jnp.float32),
        ],
        compiler_params=pltpu.CompilerParams(
            dimension_semantics=("parallel", "arbitrary")),
    )(gathered, aux, jnp.concatenate([e0, e0], axis=1), w1d, b1d,
      w2d, b2d, sel3, b3.reshape(1, 1))

    return (o_set.reshape(B, 1), o_uni.reshape(B, 1), o_prd.reshape(B, 1))


# speedup vs baseline: 3.2990x; 1.0329x over previous
import functools

import jax
import jax.numpy as jnp
from jax import lax
from jax.experimental import pallas as pl
from jax.experimental.pallas import tpu as pltpu
from jax.experimental.pallas import tpu_sc as plsc


def _make_sc_gather(num_rows, dim):
    info = plsc.get_sparse_core_info()
    nc, ns = info.num_cores, info.num_subcores
    nw = nc * ns
    assert num_rows % (nw * 256) == 0
    rows_per_w = num_rows // nw
    n_streams = 6
    while (rows_per_w // 128) % n_streams or n_streams % 2:
        n_streams -= 1
    chunk = n_streams * 128
    n_chunks = rows_per_w // chunk

    mesh = plsc.VectorSubcoreMesh(core_axis_name="c", subcore_axis_name="s")

    @functools.partial(
        pl.kernel,
        mesh=mesh,
        out_type=jax.ShapeDtypeStruct((num_rows // 2, 2 * dim), jnp.float32),
        scratch_types=[
            pltpu.VMEM((chunk,), jnp.int32),
            pltpu.VMEM((chunk, dim), jnp.float32),
            pltpu.SemaphoreType.DMA,
        ],
        compiler_params=pltpu.CompilerParams(use_tc_tiling_on_sc=False),
    )
    def gather_k(ids_hbm, table_hbm, out_hbm, idx_v, rows_v, sem):
        wid = lax.axis_index("s") * nc + lax.axis_index("c")
        base = wid * rows_per_w

        def body(g, carry):
            off = base + g * chunk
            pltpu.sync_copy(ids_hbm.at[pl.ds(off, chunk)], idx_v)
            copies = [
                pltpu.async_copy(
                    table_hbm.at[idx_v.at[pl.ds(s * 128, 128)]],
                    rows_v.at[pl.ds(s * 128, 128)],
                    sem)
                for s in range(n_streams)
            ]
            for cp in copies:
                cp.wait()
            out_rows = out_hbm.at[pl.ds(off // 2, chunk // 2)]
            writes = [
                pltpu.async_copy(
                    rows_v.at[pl.ds((u * 2 + h) * 128, 128)],
                    out_rows.at[pl.ds(u * 128, 128), pl.ds(h * dim, dim)],
                    sem)
                for u in range(n_streams // 2) for h in range(2)
            ]
            for cp in writes:
                cp.wait()
            return carry

        lax.fori_loop(0, n_chunks, body, 0)

    return gather_k


def _tc_body(x_ref, aux_ref, e0_ref, w1d_ref, b1d_ref, w2d_ref, b2d_ref,
             sel3_ref, b3_ref, o_set, o_uni, o_prd, acc,
             *, n_tok, dim, hid):
    j = pl.program_id(1)
    x2 = x_ref[...]
    w1d = w1d_ref[...]
    h = jnp.maximum(
        jnp.dot(x2.astype(jnp.bfloat16), w1d,
                preferred_element_type=jnp.float32) + b1d_ref[...], 0.0)

    @pl.when(j == 0)
    def _():
        acc[...] = h

    @pl.when(jnp.logical_and(j > 0, j < n_tok - 1))
    def _():
        acc[...] = acc[...] + h

    @pl.when(j == n_tok - 1)
    def _():
        cd = jnp.maximum(
            jnp.dot(e0_ref[...].astype(jnp.bfloat16), w1d,
                    preferred_element_type=jnp.float32) + b1d_ref[...],
            0.0)
        aux = aux_ref[...]
        lane = lax.broadcasted_iota(jnp.int32, (1, 2 * hid), 1)
        in_a = lane < hid
        n0 = jnp.where(in_a, aux[:, 0:1], aux[:, 1:2])
        i0 = jnp.where(in_a, aux[:, 2:3], aux[:, 3:4])
        set_p = acc[...] - n0 * cd
        uni_p = set_p + h - i0 * cd
        w2d = w2d_ref[...]
        b2d = b2d_ref[...]
        sel3 = sel3_ref[...]
        b3 = b3_ref[0, 0]

        def score(p):
            s = jnp.maximum(
                jnp.dot(p.astype(jnp.bfloat16), w2d,
                        preferred_element_type=jnp.float32) + b2d, 0.0)
            return jnp.dot(s.astype(jnp.bfloat16), sel3,
                           preferred_element_type=jnp.float32) + b3

        sc_s = score(set_p)
        sc_u = score(uni_p)
        o_set[...] = sc_s
        o_uni[...] = sc_u
        o_prd[...] = 1.0 / (1.0 + jnp.exp(sc_s - sc_u))


def kernel(batch_set_tensor, batch_inst_tensor, emb_table, W1, b1, W2, b2, W3, b3):
    B, L = batch_set_tensor.shape
    n_tok = L + 1
    D = emb_table.shape[1]
    H = W1.shape[1]
    T = B * n_tok

    union = jnp.concatenate([batch_set_tensor, batch_inst_tensor], axis=1)
    ids_t = union.T
    ids_flat = ids_t.reshape(T).astype(jnp.int32)
    ids_shuf = ids_flat.reshape(T // 256, 128, 2).transpose(0, 2, 1).reshape(T)

    gathered = _make_sc_gather(T, D)(ids_shuf, emb_table)

    n0s = jnp.sum(batch_set_tensor == 0, axis=1).astype(jnp.float32)
    i0 = (batch_inst_tensor[:, 0] == 0).astype(jnp.float32)
    aux = jnp.concatenate(
        [n0s.reshape(B // 2, 2), i0.reshape(B // 2, 2)], axis=1)
    e0 = emb_table[0:1, :]

    BBLK = 8192
    HB = BBLK // 2
    nb = B // BBLK

    zdh = jnp.zeros((D, H), jnp.float32)
    zhh = jnp.zeros((H, H), jnp.float32)
    zh1 = jnp.zeros((H, 1), jnp.float32)
    w1d = jnp.concatenate([
        jnp.concatenate([W1, zdh], axis=1),
        jnp.concatenate([zdh, W1], axis=1)], axis=0).astype(jnp.bfloat16)
    w2d = jnp.concatenate([
        jnp.concatenate([W2, zhh], axis=1),
        jnp.concatenate([zhh, W2], axis=1)], axis=0).astype(jnp.bfloat16)
    sel3 = jnp.concatenate([
        jnp.concatenate([W3, zh1], axis=1),
        jnp.concatenate([zh1, W3], axis=1)], axis=0).astype(jnp.bfloat16)
    b1d = jnp.concatenate([b1, b1]).reshape(1, 2 * H)
    b2d = jnp.concatenate([b2, b2]).reshape(1, 2 * H)

    out_sds = jax.ShapeDtypeStruct((B // 2, 2), jnp.float32)
    body = functools.partial(_tc_body, n_tok=n_tok, dim=D, hid=H)
    o_set, o_uni, o_prd = pl.pallas_call(
        body,
        grid=(nb, n_tok),
        in_specs=[
            pl.BlockSpec((HB, 2 * D), lambda i, j, nb=nb: (j * nb + i, 0)),
            pl.BlockSpec((HB, 4), lambda i, j: (i, 0)),
            pl.BlockSpec((1, 2 * D), lambda i, j: (0, 0)),
            pl.BlockSpec((2 * D, 2 * H), lambda i, j: (0, 0)),
            pl.BlockSpec((1, 2 * H), lambda i, j: (0, 0)),
            pl.BlockSpec((2 * H, 2 * H), lambda i, j: (0, 0)),
            pl.BlockSpec((1, 2 * H), lambda i, j: (0, 0)),
            pl.BlockSpec((2 * H, 2), lambda i, j: (0, 0)),
            pl.BlockSpec((1, 1), lambda i, j: (0, 0)),
        ],
        out_specs=[
            pl.BlockSpec((HB, 2), lambda i, j: (i, 0)),
            pl.BlockSpec((HB, 2), lambda i, j: (i, 0)),
            pl.BlockSpec((HB, 2), lambda i, j: (i, 0)),
        ],
        out_shape=[out_sds, out_sds, out_sds],
        scratch_shapes=[
            pltpu.VMEM((HB, 2 * H), jnp.float32),
        ],
        compiler_params=pltpu.CompilerParams(
            dimension_semantics=("parallel", "arbitrary")),
    )(gathered, aux, jnp.concatenate([e0, e0], axis=1), w1d, b1d,
      w2d, b2d, sel3, b3.reshape(1, 1))

    return (o_set.reshape(B, 1), o_uni.reshape(B, 1), o_prd.reshape(B, 1))
